# baseline (speedup 1.0000x reference)
import jax
import jax.numpy as jnp
from jax import lax
from jax.experimental import pallas as pl
from jax.experimental.pallas import tpu as pltpu
from jax.experimental.pallas import tpu_sc as plsc

NC = 2
NS = 16
NW = NC * NS
BATCH = 16384
BPW = BATCH // NW
D = 4
WPW = BPW * D
CH = 128
L = 16


def _rsqrt_clamped(x):
    i = lax.bitcast_convert_type(x, jnp.int32)
    i = jnp.int32(0x5F3759DF) - (i >> 1)
    y = lax.bitcast_convert_type(i, jnp.float32)
    for _ in range(3):
        y = y * (jnp.float32(1.5) - jnp.float32(0.5) * x * y * y)
    return jnp.minimum(y, jnp.float32(1e12))


def _body(uidx_hbm, iidx_hbm, utab_hbm, itab_hbm, out_hbm,
          idx_u, idx_v, widx_u, widx_v, st_u, st_v, out_v, sem):
    wid = lax.axis_index("s") * NC + lax.axis_index("c")

    pltpu.sync_copy(uidx_hbm.at[pl.ds(wid * BPW, BPW)], idx_u)
    pltpu.sync_copy(iidx_hbm.at[pl.ds(wid * BPW, BPW)], idx_v)

    for k in range(BPW // L):
        base = k * L
        wu = idx_u[pl.ds(base, L)] * D
        wv = idx_v[pl.ds(base, L)] * D
        for j in range(D):
            widx_u[pl.ds(j * BPW + base, L)] = wu + j
            widx_v[pl.ds(j * BPW + base, L)] = wv + j

    descs = []
    for c in range(WPW // CH):
        descs.append(pltpu.async_copy(
            utab_hbm.at[widx_u.at[pl.ds(c * CH, CH)]],
            st_u.at[pl.ds(c * CH, CH)], sem))
        descs.append(pltpu.async_copy(
            itab_hbm.at[widx_v.at[pl.ds(c * CH, CH)]],
            st_v.at[pl.ds(c * CH, CH)], sem))
    for d in descs:
        d.wait()

    for i in range(BPW // L):
        b = i * L
        u = [st_u[pl.ds(j * BPW + b, L)] for j in range(D)]
        v = [st_v[pl.ds(j * BPW + b, L)] for j in range(D)]
        dot = u[0] * v[0] + u[1] * v[1] + u[2] * v[2] + u[3] * v[3]
        nu = u[0] * u[0] + u[1] * u[1] + u[2] * u[2] + u[3] * u[3]
        nv = v[0] * v[0] + v[1] * v[1] + v[2] * v[2] + v[3] * v[3]
        out_v[pl.ds(b, L)] = dot * _rsqrt_clamped(nu) * _rsqrt_clamped(nv)

    pltpu.sync_copy(out_v, out_hbm.at[pl.ds(wid * BPW, BPW)])


@jax.jit
def _run(user_input, item_input, user_table, item_table):
    mesh = plsc.VectorSubcoreMesh(core_axis_name="c", subcore_axis_name="s")
    fused = pl.kernel(
        _body,
        out_type=jax.ShapeDtypeStruct((BATCH,), jnp.float32),
        mesh=mesh,
        scratch_types=[
            pltpu.VMEM((BPW,), jnp.int32),
            pltpu.VMEM((BPW,), jnp.int32),
            pltpu.VMEM((WPW,), jnp.int32),
            pltpu.VMEM((WPW,), jnp.int32),
            pltpu.VMEM((WPW,), jnp.float32),
            pltpu.VMEM((WPW,), jnp.float32),
            pltpu.VMEM((BPW,), jnp.float32),
            pltpu.SemaphoreType.DMA,
        ],
        compiler_params=pltpu.CompilerParams(needs_layout_passes=False),
    )
    return fused(user_input, item_input,
                 user_table.reshape(-1), item_table.reshape(-1))


def kernel(user_input, item_input, user_table, item_table):
    return _run(user_input, item_input, user_table, item_table)


# speedup vs baseline: 1.0009x; 1.0009x over previous
import jax
import jax.numpy as jnp
from jax import lax
from jax.experimental import pallas as pl
from jax.experimental.pallas import tpu as pltpu
from jax.experimental.pallas import tpu_sc as plsc

NC = 2
NS = 16
NW = NC * NS
BATCH = 16384
BPW = BATCH // NW
D = 4
WPW = BPW * D
CH = 128
L = 16


def _rsqrt_clamped(x):
    i = lax.bitcast_convert_type(x, jnp.int32)
    i = jnp.int32(0x5F3759DF) - (i >> 1)
    y = lax.bitcast_convert_type(i, jnp.float32)
    for _ in range(3):
        y = y * (jnp.float32(1.5) - jnp.float32(0.5) * x * y * y)
    return jnp.minimum(y, jnp.float32(1e12))


def _body(uidx_hbm, iidx_hbm, utab_hbm, itab_hbm, out_hbm,
          idx_u, idx_v, widx_u, widx_v, st_u, st_v, out_v, sem):
    wid = lax.axis_index("s") * NC + lax.axis_index("c")
    pltpu.sync_copy(uidx_hbm.at[pl.ds(wid * BPW, BPW)], idx_u)
    pltpu.sync_copy(iidx_hbm.at[pl.ds(wid * BPW, BPW)], idx_v)

    for k in range(BPW // L):
        base = k * L
        wu = idx_u[pl.ds(base, L)] * D
        wv = idx_v[pl.ds(base, L)] * D
        for j in range(D):
            widx_u[pl.ds(j * BPW + base, L)] = wu + j
            widx_v[pl.ds(j * BPW + base, L)] = wv + j

    descs = []
    for c in range(WPW // CH):
        descs.append(pltpu.async_copy(
            utab_hbm.at[widx_u.at[pl.ds(c * CH, CH)]],
            st_u.at[pl.ds(c * CH, CH)], sem))
        descs.append(pltpu.async_copy(
            itab_hbm.at[widx_v.at[pl.ds(c * CH, CH)]],
            st_v.at[pl.ds(c * CH, CH)], sem))
    for d in descs:
        d.wait()

    for i in range(BPW // L):
        b = i * L
        u = [st_u[pl.ds(j * BPW + b, L)] for j in range(D)]
        v = [st_v[pl.ds(j * BPW + b, L)] for j in range(D)]
        dot = u[0] * v[0] + u[1] * v[1] + u[2] * v[2] + u[3] * v[3]
        nu = u[0] * u[0] + u[1] * u[1] + u[2] * u[2] + u[3] * u[3]
        nv = v[0] * v[0] + v[1] * v[1] + v[2] * v[2] + v[3] * v[3]
        out_v[pl.ds(b, L)] = dot * _rsqrt_clamped(nu) * _rsqrt_clamped(nv)

    pltpu.sync_copy(out_v, out_hbm.at[pl.ds(wid * BPW, BPW)])


def _run_flat(user_input, item_input, user_flat, item_flat):
    mesh = plsc.VectorSubcoreMesh(core_axis_name="c", subcore_axis_name="s")
    fused = pl.kernel(
        _body,
        out_type=jax.ShapeDtypeStruct((BATCH,), jnp.float32),
        mesh=mesh,
        scratch_types=[
            pltpu.VMEM((BPW,), jnp.int32),
            pltpu.VMEM((BPW,), jnp.int32),
            pltpu.VMEM((WPW,), jnp.int32),
            pltpu.VMEM((WPW,), jnp.int32),
            pltpu.VMEM((WPW,), jnp.float32),
            pltpu.VMEM((WPW,), jnp.float32),
            pltpu.VMEM((BPW,), jnp.float32),
            pltpu.SemaphoreType.DMA,
        ],
        compiler_params=pltpu.CompilerParams(needs_layout_passes=False),
    )
    return fused(user_input, item_input, user_flat, item_flat)


@jax.jit
def _run(user_input, item_input, user_table, item_table):
    return _run_flat(user_input, item_input,
                     (user_table + 0.0).reshape(-1),
                     (item_table + 0.0).reshape(-1))


def kernel(user_input, item_input, user_table, item_table):
    return _run(user_input, item_input, user_table, item_table)


# speedup vs baseline: 1.0015x; 1.0007x over previous
import jax
import jax.numpy as jnp
from jax import lax
from jax.experimental import pallas as pl
from jax.experimental.pallas import tpu as pltpu
from jax.experimental.pallas import tpu_sc as plsc

NC = 2
NS = 16
NW = NC * NS
BATCH = 16384
BPW = BATCH // NW
D = 4
WPW = BPW * D
CH = 128
L = 16


def _rsqrt_clamped(x):
    i = lax.bitcast_convert_type(x, jnp.int32)
    i = jnp.int32(0x5F3759DF) - (i >> 1)
    y = lax.bitcast_convert_type(i, jnp.float32)
    for _ in range(3):
        y = y * (jnp.float32(1.5) - jnp.float32(0.5) * x * y * y)
    return jnp.minimum(y, jnp.float32(1e12))


def _body(uidx_hbm, iidx_hbm, utab_hbm, itab_hbm, out_hbm,
          idx_u, idx_v, widx_u, widx_v, st_u, st_v, out_v, sem):
    wid = lax.axis_index("s") * NC + lax.axis_index("c")
    pltpu.sync_copy(uidx_hbm.at[pl.ds(wid * BPW, BPW)], idx_u)
    pltpu.sync_copy(iidx_hbm.at[pl.ds(wid * BPW, BPW)], idx_v)

    for k in range(BPW // L):
        base = k * L
        wu = idx_u[pl.ds(base, L)] * D
        wv = idx_v[pl.ds(base, L)] * D
        for j in range(D):
            widx_u[pl.ds(j * BPW + base, L)] = wu + j
            widx_v[pl.ds(j * BPW + base, L)] = wv + j

    descs = []
    for c in range(WPW // CH):
        descs.append(pltpu.async_copy(
            utab_hbm.at[widx_u.at[pl.ds(c * CH, CH)]],
            st_u.at[pl.ds(c * CH, CH)], sem))
        descs.append(pltpu.async_copy(
            itab_hbm.at[widx_v.at[pl.ds(c * CH, CH)]],
            st_v.at[pl.ds(c * CH, CH)], sem))
    for d in descs:
        d.wait()

    for i in range(BPW // L):
        b = i * L
        u = [st_u[pl.ds(j * BPW + b, L)] for j in range(D)]
        v = [st_v[pl.ds(j * BPW + b, L)] for j in range(D)]
        dot = u[0] * v[0] + u[1] * v[1] + u[2] * v[2] + u[3] * v[3]
        nu = u[0] * u[0] + u[1] * u[1] + u[2] * u[2] + u[3] * u[3]
        nv = v[0] * v[0] + v[1] * v[1] + v[2] * v[2] + v[3] * v[3]
        out_v[pl.ds(b, L)] = dot * _rsqrt_clamped(nu) * _rsqrt_clamped(nv)

    pltpu.sync_copy(out_v, out_hbm.at[pl.ds(wid * BPW, BPW)])


def _run_flat(user_input, item_input, user_flat, item_flat):
    mesh = plsc.VectorSubcoreMesh(core_axis_name="c", subcore_axis_name="s")
    fused = pl.kernel(
        _body,
        out_type=jax.ShapeDtypeStruct((BATCH,), jnp.float32),
        mesh=mesh,
        scratch_types=[
            pltpu.VMEM((BPW,), jnp.int32),
            pltpu.VMEM((BPW,), jnp.int32),
            pltpu.VMEM((WPW,), jnp.int32),
            pltpu.VMEM((WPW,), jnp.int32),
            pltpu.VMEM((WPW,), jnp.float32),
            pltpu.VMEM((WPW,), jnp.float32),
            pltpu.VMEM((BPW,), jnp.float32),
            pltpu.SemaphoreType.DMA,
        ],
        compiler_params=pltpu.CompilerParams(needs_layout_passes=False),
    )
    return fused(user_input, item_input, user_flat, item_flat)


@jax.jit
def _run(user_input, item_input, user_table, item_table):
    return _run_flat(user_input, item_input,
                     user_table.reshape(-1), item_table.reshape(-1))


def kernel(user_input, item_input, user_table, item_table):
    return _run(user_input, item_input, user_table, item_table)
